# baseline (speedup 1.0000x reference)
import functools

import jax
import jax.numpy as jnp
import numpy as np
from jax import lax
from jax.experimental import pallas as pl
from jax.experimental.pallas import tpu as pltpu
from jax.experimental.pallas import tpu_sc as plsc

NH = 8
NL = 4
NP = 4
HD = 32
LVL = 64
L2LVL = LVL * LVL


def _vproj_body(v_ref, w_ref, b_ref, out_ref, *, blk):
    v4 = v_ref[0].astype(jnp.bfloat16).reshape(blk // 4, 4 * v_ref.shape[2])
    for h in range(NH):
        acc = jnp.dot(v4, w_ref[h], preferred_element_type=jnp.float32)
        out_ref[h, 0] = acc + b_ref[h]


def _vproj(value, vp_w, vp_b, bi, blk=2048):
    _, l2, vd = value.shape
    w = vp_w.T.reshape(vd, NH, HD).transpose(1, 0, 2).astype(jnp.bfloat16)
    eye4 = jnp.eye(4, dtype=jnp.bfloat16)
    w4 = jax.vmap(lambda wh: jnp.kron(eye4, wh))(w)
    b4 = jnp.tile(vp_b.reshape(NH, 1, HD), (1, 1, 4))
    blkr = blk // 4
    grid = (l2 // blk,)
    return pl.pallas_call(
        functools.partial(_vproj_body, blk=blk),
        grid=grid,
        in_specs=[
            pl.BlockSpec((1, blk, vd), lambda i: (bi, i, 0)),
            pl.BlockSpec((NH, 4 * vd, 128), lambda i: (0, 0, 0)),
            pl.BlockSpec((NH, 1, 128), lambda i: (0, 0, 0)),
        ],
        out_specs=pl.BlockSpec((NH, 1, blkr, 128), lambda i: (0, 0, i, 0)),
        out_shape=jax.ShapeDtypeStruct((NH, 1, l2 // 4, 128), jnp.float32),
    )(value, w4, b4)


def _samp_body(q_ref, rw_ref, swx_ref, swy_ref, sbx_ref, sby_ref, awt_ref,
               g_ref, iw_ref, aw_ref, *, l2, blk):
    q = q_ref[0]
    sx = jnp.dot(q, swx_ref[...], preferred_element_type=jnp.float32) + sbx_ref[...]
    sy = jnp.dot(q, swy_ref[...], preferred_element_type=jnp.float32) + sby_ref[...]
    logits = jnp.dot(q, awt_ref[...], preferred_element_type=jnp.float32)
    e = jnp.exp(logits)
    gs = jnp.dot(e, g_ref[...], preferred_element_type=jnp.float32)
    awn = e / gs
    aw_ref[0] = awn

    rw = rw_ref[0]
    cx = rw[:, 0:1]
    cy = rw[:, 1:2]
    ww = rw[:, 2:3]
    wh = rw[:, 3:4]
    x = cx * float(LVL) + sx * (ww * (float(LVL) / 8.0)) - 0.5
    y = cy * float(LVL) + sy * (wh * (float(LVL) / 8.0)) - 0.5
    x0f = jnp.floor(x)
    y0f = jnp.floor(y)
    fx = x - x0f
    fy = y - y0f
    x0 = x0f.astype(jnp.int32)
    y0 = y0f.astype(jnp.int32)

    lane = lax.broadcasted_iota(jnp.int32, (blk, 128), 1)
    h_lane = lane >> 4
    lvl_lane = (lane & 15) >> 2
    base = h_lane * l2 + lvl_lane * L2LVL

    for c, (dx, dy) in enumerate(((0, 0), (1, 0), (0, 1), (1, 1))):
        xi = x0 + dx
        yi = y0 + dy
        valid = ((xi >= 0) & (xi < LVL) & (yi >= 0) & (yi < LVL)).astype(jnp.float32)
        xc = jnp.clip(xi, 0, LVL - 1)
        yc = jnp.clip(yi, 0, LVL - 1)
        idx = base + (yc << 6) + xc
        wx = fx if dx else (1.0 - fx)
        wy = fy if dy else (1.0 - fy)
        wgt_i = lax.bitcast_convert_type(wx * wy * valid * awn, jnp.int32)
        for h in range(NH):
            iw_ref[h, 0, :, c * 16:(c + 1) * 16] = idx[:, h * 16:(h + 1) * 16]
            iw_ref[h, 0, :, 64 + c * 16:64 + (c + 1) * 16] = \
                wgt_i[:, h * 16:(h + 1) * 16]


def _samp_params(query, ref_windows, so_w, so_b, aw_w, l2, bi, blk=1024):
    _, l1, qd = query.shape
    swx = so_w[0::2].T
    swy = so_w[1::2].T
    sbx = so_b[0::2].reshape(1, 128)
    sby = so_b[1::2].reshape(1, 128)
    awt = aw_w.T
    g = jnp.asarray(np.kron(np.eye(NH, dtype=np.float32),
                            np.ones((16, 16), dtype=np.float32)))
    grid = (l1 // blk,)
    return pl.pallas_call(
        functools.partial(_samp_body, l2=l2, blk=blk),
        grid=grid,
        in_specs=[
            pl.BlockSpec((1, blk, qd), lambda i: (bi, i, 0)),
            pl.BlockSpec((1, blk, 4), lambda i: (bi, i, 0)),
            pl.BlockSpec((qd, 128), lambda i: (0, 0)),
            pl.BlockSpec((qd, 128), lambda i: (0, 0)),
            pl.BlockSpec((1, 128), lambda i: (0, 0)),
            pl.BlockSpec((1, 128), lambda i: (0, 0)),
            pl.BlockSpec((qd, 128), lambda i: (0, 0)),
            pl.BlockSpec((128, 128), lambda i: (0, 0)),
        ],
        out_specs=[pl.BlockSpec((NH, 1, blk, 128), lambda i: (0, 0, i, 0)),
                   pl.BlockSpec((1, blk, 128), lambda i: (0, i, 0))],
        out_shape=[jax.ShapeDtypeStruct((NH, 1, l1, 128), jnp.int32),
                   jax.ShapeDtypeStruct((1, l1, 128), jnp.float32)],
    )(query, ref_windows, swx, swy, sbx, sby, awt, g)


CQ = 16
SUPQ = 128
SUPW = SUPQ * 128


def _sc_gather(tab2, iw_all, b, l1):
    nq_tile = (b * l1 * NH) // 32
    nsup = nq_tile // SUPQ
    mesh = plsc.VectorSubcoreMesh(core_axis_name="c", subcore_axis_name="s")

    @functools.partial(
        pl.kernel,
        out_type=jax.ShapeDtypeStruct((NH, b, l1, HD), jnp.float32),
        mesh=mesh,
        scratch_types=[
            pltpu.VMEM((2, SUPQ, 128), jnp.int32),
            pltpu.VMEM((2, CQ * 64, HD), jnp.float32),
            pltpu.VMEM((2, SUPQ, HD), jnp.float32),
            pltpu.SemaphoreType.DMA,
            pltpu.SemaphoreType.DMA,
            pltpu.SemaphoreType.DMA,
            pltpu.SemaphoreType.DMA,
            pltpu.SemaphoreType.DMA,
            pltpu.SemaphoreType.DMA,
        ],
        compiler_params=pltpu.CompilerParams(use_tc_tiling_on_sc=False,
                                             needs_layout_passes=False),
    )
    def k(tab_ref, iw_hbm, out_ref,
          iw_sup, rows, out_sup, si0, si1, sg0, sg1, so0, so1):
        cid = lax.axis_index("c")
        sid = lax.axis_index("s")
        wid = sid * 2 + cid
        h = wid >> 2
        if b == 2:
            bb = (wid >> 1) & 1
            qh = wid & 1
        else:
            bb = 0
            qh = wid & 3
        qbase = qh * nq_tile

        def fire_gathers(iw_slot, qoff, rslot, sem):
            for qq in range(CQ):
                pltpu.async_copy(
                    tab_ref.at[iw_slot.at[qoff + qq, pl.ds(0, 64)]],
                    rows.at[rslot, pl.ds(qq * 64, 64)], sem)

        def drain_gathers(rslot, sem):
            pltpu.make_async_copy(tab_ref.at[pl.ds(0, CQ * 64)],
                                  rows.at[rslot], sem).wait()

        def fire_sup(s, slot, sem):
            pltpu.async_copy(iw_hbm.at[h, bb, pl.ds(qbase + s * SUPQ, SUPQ)],
                             iw_sup.at[slot], sem)

        def drain_sup(slot, sem):
            pltpu.make_async_copy(iw_hbm.at[h, bb, pl.ds(0, SUPQ)],
                                  iw_sup.at[slot], sem).wait()

        pltpu.sync_copy(iw_hbm.at[h, bb, pl.ds(qbase, SUPQ)], iw_sup.at[0])
        fire_gathers(iw_sup.at[0], 0, 0, sg0)

        def sg_body(sg, carry):
            for sp in range(2):
                s = sg * 2 + sp
                sp2 = 1 - sp
                si_nxt = (si0, si1)[sp2]
                so_cur = (so0, so1)[sp]

                @pl.when(s < nsup - 1)
                def _():
                    fire_sup(s + 1, sp2, si_nxt)

                @pl.when(s >= 2)
                def _():
                    pltpu.make_async_copy(
                        out_sup.at[sp],
                        out_ref.at[h, bb, pl.ds(qbase, SUPQ)], so_cur).wait()

                def jj_body(jj, carry2):
                    for cp in range(2):
                        j = jj * 2 + cp
                        if cp == 0:
                            fire_gathers(iw_sup.at[sp], (j + 1) * CQ, 1, sg1)
                        else:
                            @pl.when(jj < 3)
                            def _():
                                fire_gathers(iw_sup.at[sp], (j + 1) * CQ, 0,
                                             sg0)

                            @pl.when((jj == 3) & (s < nsup - 1))
                            def _():
                                drain_sup(sp2, si_nxt)
                                fire_gathers(iw_sup.at[sp2], 0, 0, sg0)

                        drain_gathers(cp, (sg0, sg1)[cp])

                        def q_body(qi, carry3):
                            z = jnp.zeros((16,), jnp.float32)
                            a0 = z
                            a1 = z
                            qb = qi * 64
                            qrow = j * CQ + qi
                            for c in range(4):
                                wv = plsc.bitcast(
                                    iw_sup[sp, qrow, pl.ds(64 + c * 16, 16)],
                                    jnp.float32)
                                for lp in range(16):
                                    w = wv[lp]
                                    r = qb + c * 16 + lp
                                    v0 = rows[cp, r, pl.ds(0, 16)]
                                    v1 = rows[cp, r, pl.ds(16, 16)]
                                    a0 = a0 + w * v0
                                    a1 = a1 + w * v1
                            out_sup[sp, qrow, pl.ds(0, 16)] = a0
                            out_sup[sp, qrow, pl.ds(16, 16)] = a1
                            return carry3

                        lax.fori_loop(0, CQ, q_body, 0)
                    return carry2

                lax.fori_loop(0, 4, jj_body, 0)
                pltpu.async_copy(out_sup.at[sp],
                                 out_ref.at[h, bb, pl.ds(qbase + s * SUPQ, SUPQ)],
                                 so_cur)
            return carry

        lax.fori_loop(0, nsup // 2, sg_body, 0)
        pltpu.make_async_copy(out_sup.at[0],
                              out_ref.at[h, bb, pl.ds(qbase, SUPQ)], so0).wait()
        pltpu.make_async_copy(out_sup.at[1],
                              out_ref.at[h, bb, pl.ds(qbase, SUPQ)], so1).wait()

    return k(tab2, iw_all)


def _oproj_body(s_ref, w_ref, b_ref, out_ref):
    acc = b_ref[...].astype(jnp.float32)
    acc = jnp.broadcast_to(acc, out_ref.shape[1:])
    for h in range(NH):
        acc = acc + jnp.dot(s_ref[h, 0], w_ref[h],
                            preferred_element_type=jnp.float32)
    out_ref[0] = acc


def _oproj(samp, op_w, op_b, blk=1024):
    _, b, l1, _ = samp.shape
    qd = op_w.shape[0]
    w = op_w.T.reshape(NH, HD, qd)
    bias = op_b.reshape(1, qd)
    grid = (b, l1 // blk)
    return pl.pallas_call(
        _oproj_body,
        grid=grid,
        in_specs=[
            pl.BlockSpec((NH, 1, blk, HD), lambda bi, i: (0, bi, i, 0)),
            pl.BlockSpec((NH, HD, qd), lambda bi, i: (0, 0, 0)),
            pl.BlockSpec((1, qd), lambda bi, i: (0, 0)),
        ],
        out_specs=pl.BlockSpec((1, blk, qd), lambda bi, i: (bi, i, 0)),
        out_shape=jax.ShapeDtypeStruct((b, l1, qd), jnp.float32),
    )(samp, w, bias)


def kernel(query, value, v_shape, v_mask, v_start_index, v_valid_ratios,
           ref_windows, so_w, so_b, aw_w, aw_b, vp_w, vp_b, op_w, op_b):
    b, l1, _ = query.shape
    l2 = value.shape[1]

    outs, awns = [], []
    for bi in range(b):
        tab = _vproj(value, vp_w, vp_b, bi)
        iw, awn = _samp_params(query, ref_windows, so_w, so_b, aw_w, l2, bi)
        samp = _sc_gather(tab.reshape(NH * l2, HD), iw, 1, l1)
        outs.append(_oproj(samp, op_w, op_b))
        awns.append(awn)
    out = jnp.concatenate(outs, 0) if b > 1 else outs[0]
    awn = jnp.concatenate(awns, 0) if b > 1 else awns[0]
    aw = awn.reshape(b, l1, NH, NL, NP)
    return out, aw


# speedup vs baseline: 1.0127x; 1.0127x over previous
import functools

import jax
import jax.numpy as jnp
import numpy as np
from jax import lax
from jax.experimental import pallas as pl
from jax.experimental.pallas import tpu as pltpu
from jax.experimental.pallas import tpu_sc as plsc

NH = 8
NL = 4
NP = 4
HD = 32
LVL = 64
L2LVL = LVL * LVL


def _vproj_body(v_ref, w_ref, b_ref, out_ref, *, blk):
    v4 = v_ref[0].astype(jnp.bfloat16).reshape(blk // 4, 4 * v_ref.shape[2])
    for h in range(NH):
        acc = jnp.dot(v4, w_ref[h], preferred_element_type=jnp.float32)
        out_ref[h, 0] = acc + b_ref[h]


def _vproj(value, vp_w, vp_b, bi, blk=2048):
    _, l2, vd = value.shape
    w = vp_w.T.reshape(vd, NH, HD).transpose(1, 0, 2).astype(jnp.bfloat16)
    eye4 = jnp.eye(4, dtype=jnp.bfloat16)
    w4 = jax.vmap(lambda wh: jnp.kron(eye4, wh))(w)
    b4 = jnp.tile(vp_b.reshape(NH, 1, HD), (1, 1, 4))
    blkr = blk // 4
    grid = (l2 // blk,)
    return pl.pallas_call(
        functools.partial(_vproj_body, blk=blk),
        grid=grid,
        in_specs=[
            pl.BlockSpec((1, blk, vd), lambda i: (bi, i, 0)),
            pl.BlockSpec((NH, 4 * vd, 128), lambda i: (0, 0, 0)),
            pl.BlockSpec((NH, 1, 128), lambda i: (0, 0, 0)),
        ],
        out_specs=pl.BlockSpec((NH, 1, blkr, 128), lambda i: (0, 0, i, 0)),
        out_shape=jax.ShapeDtypeStruct((NH, 1, l2 // 4, 128), jnp.float32),
    )(value, w4, b4)


def _samp_body(q_ref, rw_ref, swx_ref, swy_ref, sbx_ref, sby_ref, awt_ref,
               g_ref, iw_ref, aw_ref, *, l2, blk):
    q = q_ref[0]
    sx = jnp.dot(q, swx_ref[...], preferred_element_type=jnp.float32) + sbx_ref[...]
    sy = jnp.dot(q, swy_ref[...], preferred_element_type=jnp.float32) + sby_ref[...]
    logits = jnp.dot(q, awt_ref[...], preferred_element_type=jnp.float32)
    e = jnp.exp(logits)
    gs = jnp.dot(e, g_ref[...], preferred_element_type=jnp.float32)
    awn = e / gs
    aw_ref[0] = awn

    rw = rw_ref[0]
    cx = rw[:, 0:1]
    cy = rw[:, 1:2]
    ww = rw[:, 2:3]
    wh = rw[:, 3:4]
    x = cx * float(LVL) + sx * (ww * (float(LVL) / 8.0)) - 0.5
    y = cy * float(LVL) + sy * (wh * (float(LVL) / 8.0)) - 0.5
    x0f = jnp.floor(x)
    y0f = jnp.floor(y)
    fx = x - x0f
    fy = y - y0f
    x0 = x0f.astype(jnp.int32)
    y0 = y0f.astype(jnp.int32)

    lane = lax.broadcasted_iota(jnp.int32, (blk, 128), 1)
    h_lane = lane >> 4
    lvl_lane = (lane & 15) >> 2
    base = h_lane * l2 + lvl_lane * L2LVL

    for c, (dx, dy) in enumerate(((0, 0), (1, 0), (0, 1), (1, 1))):
        xi = x0 + dx
        yi = y0 + dy
        valid = ((xi >= 0) & (xi < LVL) & (yi >= 0) & (yi < LVL)).astype(jnp.float32)
        xc = jnp.clip(xi, 0, LVL - 1)
        yc = jnp.clip(yi, 0, LVL - 1)
        idx = base + (yc << 6) + xc
        wx = fx if dx else (1.0 - fx)
        wy = fy if dy else (1.0 - fy)
        wgt_i = lax.bitcast_convert_type(wx * wy * valid * awn, jnp.int32)
        for h in range(NH):
            iw_ref[h, 0, :, c * 16:(c + 1) * 16] = idx[:, h * 16:(h + 1) * 16]
            iw_ref[h, 0, :, 64 + c * 16:64 + (c + 1) * 16] = \
                wgt_i[:, h * 16:(h + 1) * 16]


def _samp_params(query, ref_windows, so_w, so_b, aw_w, l2, bi, blk=512):
    _, l1, qd = query.shape
    swx = so_w[0::2].T
    swy = so_w[1::2].T
    sbx = so_b[0::2].reshape(1, 128)
    sby = so_b[1::2].reshape(1, 128)
    awt = aw_w.T
    g = jnp.asarray(np.kron(np.eye(NH, dtype=np.float32),
                            np.ones((16, 16), dtype=np.float32)))
    grid = (l1 // blk,)
    return pl.pallas_call(
        functools.partial(_samp_body, l2=l2, blk=blk),
        grid=grid,
        in_specs=[
            pl.BlockSpec((1, blk, qd), lambda i: (bi, i, 0)),
            pl.BlockSpec((1, blk, 4), lambda i: (bi, i, 0)),
            pl.BlockSpec((qd, 128), lambda i: (0, 0)),
            pl.BlockSpec((qd, 128), lambda i: (0, 0)),
            pl.BlockSpec((1, 128), lambda i: (0, 0)),
            pl.BlockSpec((1, 128), lambda i: (0, 0)),
            pl.BlockSpec((qd, 128), lambda i: (0, 0)),
            pl.BlockSpec((128, 128), lambda i: (0, 0)),
        ],
        out_specs=[pl.BlockSpec((NH, 1, blk, 128), lambda i: (0, 0, i, 0)),
                   pl.BlockSpec((1, blk, 128), lambda i: (0, i, 0))],
        out_shape=[jax.ShapeDtypeStruct((NH, 1, l1, 128), jnp.int32),
                   jax.ShapeDtypeStruct((1, l1, 128), jnp.float32)],
    )(query, ref_windows, swx, swy, sbx, sby, awt, g)


CQ = 16
SUPQ = 128
SUPW = SUPQ * 128


def _sc_gather(tab2, iw_all, b, l1):
    nq_tile = (b * l1 * NH) // 32
    nsup = nq_tile // SUPQ
    mesh = plsc.VectorSubcoreMesh(core_axis_name="c", subcore_axis_name="s")

    @functools.partial(
        pl.kernel,
        out_type=jax.ShapeDtypeStruct((NH, b, l1, HD), jnp.float32),
        mesh=mesh,
        scratch_types=[
            pltpu.VMEM((2, SUPQ, 128), jnp.int32),
            pltpu.VMEM((2, CQ * 64, HD), jnp.float32),
            pltpu.VMEM((2, SUPQ, HD), jnp.float32),
            pltpu.SemaphoreType.DMA,
            pltpu.SemaphoreType.DMA,
            pltpu.SemaphoreType.DMA,
            pltpu.SemaphoreType.DMA,
            pltpu.SemaphoreType.DMA,
            pltpu.SemaphoreType.DMA,
        ],
        compiler_params=pltpu.CompilerParams(use_tc_tiling_on_sc=False,
                                             needs_layout_passes=False),
    )
    def k(tab_ref, iw_hbm, out_ref,
          iw_sup, rows, out_sup, si0, si1, sg0, sg1, so0, so1):
        cid = lax.axis_index("c")
        sid = lax.axis_index("s")
        wid = sid * 2 + cid
        h = wid >> 2
        if b == 2:
            bb = (wid >> 1) & 1
            qh = wid & 1
        else:
            bb = 0
            qh = wid & 3
        qbase = qh * nq_tile

        def fire_gathers(iw_slot, qoff, rslot, sem):
            for qq in range(CQ):
                pltpu.async_copy(
                    tab_ref.at[iw_slot.at[qoff + qq, pl.ds(0, 64)]],
                    rows.at[rslot, pl.ds(qq * 64, 64)], sem)

        def drain_gathers(rslot, sem):
            pltpu.make_async_copy(tab_ref.at[pl.ds(0, CQ * 64)],
                                  rows.at[rslot], sem).wait()

        def fire_sup(s, slot, sem):
            pltpu.async_copy(iw_hbm.at[h, bb, pl.ds(qbase + s * SUPQ, SUPQ)],
                             iw_sup.at[slot], sem)

        def drain_sup(slot, sem):
            pltpu.make_async_copy(iw_hbm.at[h, bb, pl.ds(0, SUPQ)],
                                  iw_sup.at[slot], sem).wait()

        pltpu.sync_copy(iw_hbm.at[h, bb, pl.ds(qbase, SUPQ)], iw_sup.at[0])
        fire_gathers(iw_sup.at[0], 0, 0, sg0)

        def sg_body(sg, carry):
            for sp in range(2):
                s = sg * 2 + sp
                sp2 = 1 - sp
                si_nxt = (si0, si1)[sp2]
                so_cur = (so0, so1)[sp]

                @pl.when(s < nsup - 1)
                def _():
                    fire_sup(s + 1, sp2, si_nxt)

                @pl.when(s >= 2)
                def _():
                    pltpu.make_async_copy(
                        out_sup.at[sp],
                        out_ref.at[h, bb, pl.ds(qbase, SUPQ)], so_cur).wait()

                def jj_body(jj, carry2):
                    for cp in range(2):
                        j = jj * 2 + cp
                        if cp == 0:
                            fire_gathers(iw_sup.at[sp], (j + 1) * CQ, 1, sg1)
                        else:
                            @pl.when(jj < 3)
                            def _():
                                fire_gathers(iw_sup.at[sp], (j + 1) * CQ, 0,
                                             sg0)

                            @pl.when((jj == 3) & (s < nsup - 1))
                            def _():
                                drain_sup(sp2, si_nxt)
                                fire_gathers(iw_sup.at[sp2], 0, 0, sg0)

                        drain_gathers(cp, (sg0, sg1)[cp])

                        def q_body(qi, carry3):
                            z = jnp.zeros((16,), jnp.float32)
                            a0 = z
                            a1 = z
                            qb = qi * 64
                            qrow = j * CQ + qi
                            for c in range(4):
                                wv = plsc.bitcast(
                                    iw_sup[sp, qrow, pl.ds(64 + c * 16, 16)],
                                    jnp.float32)
                                for lp in range(16):
                                    w = wv[lp]
                                    r = qb + c * 16 + lp
                                    v0 = rows[cp, r, pl.ds(0, 16)]
                                    v1 = rows[cp, r, pl.ds(16, 16)]
                                    a0 = a0 + w * v0
                                    a1 = a1 + w * v1
                            out_sup[sp, qrow, pl.ds(0, 16)] = a0
                            out_sup[sp, qrow, pl.ds(16, 16)] = a1
                            return carry3

                        lax.fori_loop(0, CQ, q_body, 0)
                    return carry2

                lax.fori_loop(0, 4, jj_body, 0)
                pltpu.async_copy(out_sup.at[sp],
                                 out_ref.at[h, bb, pl.ds(qbase + s * SUPQ, SUPQ)],
                                 so_cur)
            return carry

        lax.fori_loop(0, nsup // 2, sg_body, 0)
        pltpu.make_async_copy(out_sup.at[0],
                              out_ref.at[h, bb, pl.ds(qbase, SUPQ)], so0).wait()
        pltpu.make_async_copy(out_sup.at[1],
                              out_ref.at[h, bb, pl.ds(qbase, SUPQ)], so1).wait()

    return k(tab2, iw_all)


def _oproj_body(s_ref, w_ref, b_ref, out_ref):
    acc = b_ref[...].astype(jnp.float32)
    acc = jnp.broadcast_to(acc, out_ref.shape[1:])
    for h in range(NH):
        acc = acc + jnp.dot(s_ref[h, 0], w_ref[h],
                            preferred_element_type=jnp.float32)
    out_ref[0] = acc


def _oproj(samp, op_w, op_b, blk=1024):
    _, b, l1, _ = samp.shape
    qd = op_w.shape[0]
    w = op_w.T.reshape(NH, HD, qd)
    bias = op_b.reshape(1, qd)
    grid = (b, l1 // blk)
    return pl.pallas_call(
        _oproj_body,
        grid=grid,
        in_specs=[
            pl.BlockSpec((NH, 1, blk, HD), lambda bi, i: (0, bi, i, 0)),
            pl.BlockSpec((NH, HD, qd), lambda bi, i: (0, 0, 0)),
            pl.BlockSpec((1, qd), lambda bi, i: (0, 0)),
        ],
        out_specs=pl.BlockSpec((1, blk, qd), lambda bi, i: (bi, i, 0)),
        out_shape=jax.ShapeDtypeStruct((b, l1, qd), jnp.float32),
    )(samp, w, bias)


def kernel(query, value, v_shape, v_mask, v_start_index, v_valid_ratios,
           ref_windows, so_w, so_b, aw_w, aw_b, vp_w, vp_b, op_w, op_b):
    b, l1, _ = query.shape
    l2 = value.shape[1]

    outs, awns = [], []
    for bi in range(b):
        tab = _vproj(value, vp_w, vp_b, bi)
        iw, awn = _samp_params(query, ref_windows, so_w, so_b, aw_w, l2, bi)
        samp = _sc_gather(tab.reshape(NH * l2, HD), iw, 1, l1)
        outs.append(_oproj(samp, op_w, op_b))
        awns.append(awn)
    out = jnp.concatenate(outs, 0) if b > 1 else outs[0]
    awn = jnp.concatenate(awns, 0) if b > 1 else awns[0]
    aw = awn.reshape(b, l1, NH, NL, NP)
    return out, aw
